# baseline (speedup 1.0000x reference)
import jax
import jax.numpy as jnp
from jax import lax
from jax.experimental import pallas as pl
from jax.experimental.pallas import tpu as pltpu
from jax.experimental.pallas import tpu_sc as plsc

S = 4096
LAT = 1024
SPLIT = 3072
N = S * LAT
SPLIT_E = SPLIT * LAT
CUT = 2048
CUT_E = CUT * LAT
NW = 32
P = (N - CUT_E) // NW
NCH = 4
C = P // NCH
NBUF = 2
W = 128
G = LAT // W
SR = S * G
R = 4096
TC_BLOCKS = CUT * G // R


def _sc_body(hist_ref, hval_ref, ctr_ref, out_ref,
             b0, b1, scal_v, hv, acc, row, r0s, r1s, w0s, w1s):
    bufs = (b0, b1)
    rsem = (r0s, r1s)
    wsem = (w0s, w1s)
    cid = lax.axis_index("c")
    sid = lax.axis_index("s")
    wid = cid * 16 + sid

    pltpu.sync_copy(ctr_ref, scal_v.at[pl.ds(0, 8)])
    ctr = scal_v[pl.ds(0, 16)][0]
    ovf = ctr == SPLIT

    def offsets(c):
        d0 = CUT_E + wid * P + c * C
        is_mean = jnp.logical_and(d0 == SPLIT_E, ovf)
        shifted = jnp.logical_or(d0 < SPLIT_E, ovf)
        adj = jnp.where(
            jnp.logical_and(shifted, jnp.logical_not(is_mean)), LAT, 0
        )
        return d0 - adj, d0 + jnp.where(is_mean, LAT, 0)

    offs = [offsets(c) for c in range(NCH)]
    rh = [None] * NCH
    wh = [None] * NCH
    for c in range(NBUF):
        rh[c] = pltpu.async_copy(
            hist_ref.at[pl.ds(offs[c][0], C)], bufs[c], rsem[c]
        )
    for c in range(NCH):
        b = c % NBUF
        if c >= NBUF:
            wh[c - NBUF].wait()
            rh[c] = pltpu.async_copy(
                hist_ref.at[pl.ds(offs[c][0], C)], bufs[b], rsem[b]
            )
        rh[c].wait()
        wh[c] = pltpu.async_copy(
            bufs[b], out_ref.at[pl.ds(offs[c][1], C)], wsem[b]
        )
    for c in range(NCH - NBUF, NCH):
        wh[c].wait()

    @pl.when(jnp.logical_and(wid == (SPLIT_E - CUT_E) // P, ovf))
    def _():
        pltpu.sync_copy(hval_ref, hv)

        def initj(j, _):
            acc[pl.ds(j * 16, 16)] = hv[pl.ds(j * 16, 16)]
            return 0

        lax.fori_loop(0, LAT // 16, initj, 0)

        def body(r, _):
            pltpu.sync_copy(hist_ref.at[pl.ds(r * LAT, LAT)], row)

            def addj(j, _):
                acc[pl.ds(j * 16, 16)] = (
                    acc[pl.ds(j * 16, 16)] + row[pl.ds(j * 16, 16)]
                )
                return 0

            lax.fori_loop(0, LAT // 16, addj, 0)
            return 0

        lax.fori_loop(0, SPLIT - 1, body, 0)

        def finj(j, _):
            row[pl.ds(j * 16, 16)] = acc[pl.ds(j * 16, 16)] * (1.0 / SPLIT)
            return 0

        lax.fori_loop(0, LAT // 16, finj, 0)
        pltpu.sync_copy(row, out_ref.at[pl.ds(SPLIT_E, LAT)])


def _sc_call(histf, hvalf, ctr8):
    mesh = plsc.VectorSubcoreMesh(
        core_axis_name="c", subcore_axis_name="s", num_cores=2, num_subcores=16
    )
    k = pl.kernel(
        _sc_body,
        out_type=jax.ShapeDtypeStruct((N,), jnp.float32),
        mesh=mesh,
        scratch_types=[
            pltpu.VMEM((C,), jnp.float32),
            pltpu.VMEM((C,), jnp.float32),
            pltpu.VMEM((32,), jnp.int32),
            pltpu.VMEM((LAT,), jnp.float32),
            pltpu.VMEM((LAT,), jnp.float32),
            pltpu.VMEM((LAT,), jnp.float32),
            pltpu.SemaphoreType.DMA,
            pltpu.SemaphoreType.DMA,
            pltpu.SemaphoreType.DMA,
            pltpu.SemaphoreType.DMA,
        ],
    )
    return k(histf, hvalf, ctr8)


def _tc_body(sc_ref, hist_ref, hval_ref, out_ref, carry):
    i = pl.program_id(0)

    blk = hist_ref[...]
    first = jnp.where(i == 0, hval_ref[...], carry[...])
    out_ref[0:G, :] = first
    out_ref[G:R, :] = blk[0 : R - G, :]
    carry[...] = blk[R - G : R, :]


def _tc_call(sc_out, histr, hvalr):
    return pl.pallas_call(
        _tc_body,
        grid=(TC_BLOCKS,),
        in_specs=[
            pl.BlockSpec(memory_space=pl.ANY),
            pl.BlockSpec((R, W), lambda i: (i, 0)),
            pl.BlockSpec((G, W), lambda i: (0, 0)),
        ],
        out_specs=pl.BlockSpec((R, W), lambda i: (i, 0)),
        scratch_shapes=[pltpu.VMEM((G, W), jnp.float32)],
        out_shape=jax.ShapeDtypeStruct((SR, W), jnp.float32),
        input_output_aliases={0: 0},
    )(sc_out, histr, hvalr)


def kernel(hist, hist_time, hist_val, hist_time_val, counter, index):
    histf = hist.reshape(-1)
    sc_out = _sc_call(histf, hist_val.reshape(-1), counter.reshape(-1))
    histr = hist.reshape(hist.shape[0] * SR, W)
    out = _tc_call(sc_out.reshape(SR, W), histr, hist_val.reshape(G, W))
    return out.reshape(S, 1, LAT)


# speedup vs baseline: 1.0903x; 1.0903x over previous
import jax
import jax.numpy as jnp
from jax import lax
from jax.experimental import pallas as pl
from jax.experimental.pallas import tpu as pltpu
from jax.experimental.pallas import tpu_sc as plsc

S = 4096
LAT = 1024
SPLIT = 3072
N = S * LAT
SPLIT_E = SPLIT * LAT
CUT = 2048
CUT_E = CUT * LAT
NW = 32
P = (N - CUT_E) // NW
C = P
W = 128
G = LAT // W
SR = S * G
R = 8192
TC_BLOCKS = CUT * G // R


def _sc_body(hist_ref, hval_ref, ctr_ref, out_ref, buf, scal_v, hv, acc, row):
    cid = lax.axis_index("c")
    sid = lax.axis_index("s")
    wid = cid * 16 + sid

    pltpu.sync_copy(ctr_ref, scal_v.at[pl.ds(0, 8)])
    ctr = scal_v[pl.ds(0, 16)][0]
    ovf = ctr == SPLIT

    d0 = CUT_E + wid * P
    is_mean = jnp.logical_and(d0 == SPLIT_E, ovf)
    shifted = jnp.logical_or(d0 < SPLIT_E, ovf)
    adj = jnp.where(jnp.logical_and(shifted, jnp.logical_not(is_mean)), LAT, 0)
    src0 = d0 - adj
    dst0 = d0 + jnp.where(is_mean, LAT, 0)
    pltpu.sync_copy(hist_ref.at[pl.ds(src0, C)], buf)
    pltpu.sync_copy(buf, out_ref.at[pl.ds(dst0, C)])

    @pl.when(jnp.logical_and(wid == (SPLIT_E - CUT_E) // P, ovf))
    def _():
        pltpu.sync_copy(hval_ref, hv)

        def initj(j, _):
            acc[pl.ds(j * 16, 16)] = hv[pl.ds(j * 16, 16)]
            return 0

        lax.fori_loop(0, LAT // 16, initj, 0)

        def body(r, _):
            pltpu.sync_copy(hist_ref.at[pl.ds(r * LAT, LAT)], row)

            def addj(j, _):
                acc[pl.ds(j * 16, 16)] = (
                    acc[pl.ds(j * 16, 16)] + row[pl.ds(j * 16, 16)]
                )
                return 0

            lax.fori_loop(0, LAT // 16, addj, 0)
            return 0

        lax.fori_loop(0, SPLIT - 1, body, 0)

        def finj(j, _):
            row[pl.ds(j * 16, 16)] = acc[pl.ds(j * 16, 16)] * (1.0 / SPLIT)
            return 0

        lax.fori_loop(0, LAT // 16, finj, 0)
        pltpu.sync_copy(row, out_ref.at[pl.ds(SPLIT_E, LAT)])


def _sc_call(histf, hvalf, ctr8):
    mesh = plsc.VectorSubcoreMesh(
        core_axis_name="c", subcore_axis_name="s", num_cores=2, num_subcores=16
    )
    k = pl.kernel(
        _sc_body,
        out_type=jax.ShapeDtypeStruct((N,), jnp.float32),
        mesh=mesh,
        scratch_types=[
            pltpu.VMEM((C,), jnp.float32),
            pltpu.VMEM((32,), jnp.int32),
            pltpu.VMEM((LAT,), jnp.float32),
            pltpu.VMEM((LAT,), jnp.float32),
            pltpu.VMEM((LAT,), jnp.float32),
        ],
    )
    return k(histf, hvalf, ctr8)


def _tc_body(sc_ref, hist_ref, hval_ref, out_ref, carry):
    i = pl.program_id(0)

    blk = hist_ref[...]
    first = jnp.where(i == 0, hval_ref[...], carry[...])
    out_ref[0:G, :] = first
    out_ref[G:R, :] = blk[0 : R - G, :]
    carry[...] = blk[R - G : R, :]


def _tc_call(sc_out, histr, hvalr):
    return pl.pallas_call(
        _tc_body,
        grid=(TC_BLOCKS,),
        in_specs=[
            pl.BlockSpec(memory_space=pl.ANY),
            pl.BlockSpec((R, W), lambda i: (i, 0)),
            pl.BlockSpec((G, W), lambda i: (0, 0)),
        ],
        out_specs=pl.BlockSpec((R, W), lambda i: (i, 0)),
        scratch_shapes=[pltpu.VMEM((G, W), jnp.float32)],
        out_shape=jax.ShapeDtypeStruct((SR, W), jnp.float32),
        input_output_aliases={0: 0},
    )(sc_out, histr, hvalr)


def kernel(hist, hist_time, hist_val, hist_time_val, counter, index):
    histf = hist.reshape(-1)
    sc_out = _sc_call(histf, hist_val.reshape(-1), counter.reshape(-1))
    histr = hist.reshape(hist.shape[0] * SR, W)
    out = _tc_call(sc_out.reshape(SR, W), histr, hist_val.reshape(G, W))
    return out.reshape(S, 1, LAT)
